# baseline (speedup 1.0000x reference)
import functools

import jax
import jax.numpy as jnp
from jax import lax
from jax.experimental import pallas as pl
from jax.experimental.pallas import tpu as pltpu
from jax.experimental.pallas import tpu_sc as plsc

FREEZE_RATIO = 0.9

R = 16384
D = 1024
T = int(R * FREEZE_RATIO)
NC = 2
NS = 16
NW = NC * NS
LANES = 16
GRP = 8

GRP_LO = (T // GRP) * GRP
NZG = T - GRP_LO

COPY_LO = GRP_LO + GRP
NGROUPS = (R - COPY_LO) // GRP
GPW = NGROUPS // NW
NEXTRA = NGROUPS - GPW * NW
BASE_ROWS = GPW * GRP

ZBLK = 776
ZGRID = GRP_LO // ZBLK


_mesh = plsc.VectorSubcoreMesh(core_axis_name="c", subcore_axis_name="s")


@functools.partial(
    pl.kernel,
    mesh=_mesh,
    out_type=jax.ShapeDtypeStruct((R, D), jnp.float32),
    scratch_types=[
        pltpu.VMEM((BASE_ROWS + GRP, D), jnp.float32),
        pltpu.VMEM((GRP, D), jnp.float32),
        pltpu.SemaphoreType.DMA,
        pltpu.SemaphoreType.DMA,
    ],
)
def _tail_sc(tokens_hbm, out_hbm, buf, buf_m, sem_i, sem_o):
    wid = lax.axis_index("s") * NC + lax.axis_index("c")

    row0 = COPY_LO + (wid * GPW + jnp.minimum(wid, NEXTRA)) * GRP

    in_a = pltpu.async_copy(
        tokens_hbm.at[pl.ds(row0, BASE_ROWS)],
        buf.at[pl.ds(0, BASE_ROWS)], sem_i)

    @pl.when(wid < NEXTRA)
    def _fire_in_extra():
        pltpu.async_copy(tokens_hbm.at[pl.ds(row0 + BASE_ROWS, GRP)],
                         buf.at[pl.ds(BASE_ROWS, GRP)], sem_i)

    @pl.when(wid == NW - 1)
    def _fire_in_mixed():
        pltpu.async_copy(tokens_hbm.at[pl.ds(GRP_LO, GRP)], buf_m, sem_i)

    in_a.wait()
    out_a = pltpu.async_copy(
        buf.at[pl.ds(0, BASE_ROWS)],
        out_hbm.at[pl.ds(row0, BASE_ROWS)], sem_o)

    @pl.when(wid < NEXTRA)
    def _flush_extra():
        pltpu.make_async_copy(tokens_hbm.at[pl.ds(row0 + BASE_ROWS, GRP)],
                              buf.at[pl.ds(BASE_ROWS, GRP)], sem_i).wait()
        pltpu.async_copy(buf.at[pl.ds(BASE_ROWS, GRP)],
                         out_hbm.at[pl.ds(row0 + BASE_ROWS, GRP)],
                         sem_o).wait()

    @pl.when(wid == NW - 1)
    def _flush_mixed():
        pltpu.make_async_copy(tokens_hbm.at[pl.ds(GRP_LO, GRP)],
                              buf_m, sem_i).wait()

        def zero_col(c, carry):
            for r in range(NZG):
                buf_m[r, pl.ds(c * LANES, LANES)] = jnp.zeros(
                    (LANES,), jnp.float32)
            return carry

        lax.fori_loop(0, D // LANES, zero_col, 0)
        pltpu.async_copy(buf_m, out_hbm.at[pl.ds(GRP_LO, GRP)],
                         sem_o).wait()

    out_a.wait()


def _zero_prefix_body(x_hbm, o_ref):
    del x_hbm
    o_ref[...] = jnp.zeros_like(o_ref)


_zero_prefix = pl.pallas_call(
    _zero_prefix_body,
    grid=(ZGRID,),
    in_specs=[pl.BlockSpec(memory_space=pl.ANY)],
    out_specs=pl.BlockSpec((ZBLK, D), lambda i: (i, 0)),
    out_shape=jax.ShapeDtypeStruct((R, D), jnp.float32),
    input_output_aliases={0: 0},
)


@jax.jit
def kernel(tokens):
    b, s, d = tokens.shape
    flat = tokens.reshape(b * s, d)
    tail = _tail_sc(flat)
    out_flat = _zero_prefix(tail)
    return out_flat.reshape(b, s, d), jnp.full((), T, jnp.int32)
